# baseline (speedup 1.0000x reference)
import jax
import jax.numpy as jnp
from jax import lax
from jax.experimental import pallas as pl
from jax.experimental.pallas import tpu as pltpu
from jax.experimental.pallas import tpu_sc as plsc

_B = 4096
_D = 128
_V = 100000

_info = plsc.get_sparse_core_info()
_NC, _NS = _info.num_cores, _info.num_subcores
_NW = _NC * _NS
_BPW = _B // _NW

_BV = 1000
_GV = _V // _BV


def _gather_body(idx_hbm, table_hbm, out_hbm, idx_v, rows_v, sem):
    wid = lax.axis_index("s") * _NC + lax.axis_index("c")
    base = wid * _BPW
    pltpu.sync_copy(idx_hbm.at[pl.ds(base, _BPW)], idx_v)
    pltpu.async_copy(table_hbm.at[idx_v], rows_v, sem).wait()
    pltpu.sync_copy(rows_v, out_hbm.at[pl.ds(base, _BPW)])


_gather = pl.kernel(
    _gather_body,
    out_type=jax.ShapeDtypeStruct((_B, _D), jnp.float32),
    mesh=plsc.VectorSubcoreMesh(core_axis_name="c", subcore_axis_name="s"),
    scratch_types=[
        pltpu.VMEM((_BPW,), jnp.int32),
        pltpu.VMEM((_BPW, _D), jnp.float32),
        pltpu.SemaphoreType.DMA,
    ],
)


_NBUF = 3


def _proj_body(w_ref, emb_ref, b_ref, out_ref, buf, sems):
    v = pl.program_id(0)
    slot = lax.rem(v, _NBUF)

    def _copy(s, vv):
        return pltpu.make_async_copy(
            buf.at[s],
            out_ref.at[pl.ds(vv * _BV, _BV), :],
            sems.at[s],
        )

    @pl.when(v >= _NBUF)
    def _():
        _copy(slot, v).wait()

    w = w_ref[...].astype(jnp.bfloat16)
    e = emb_ref[...].astype(jnp.bfloat16)
    acc = lax.dot_general(w, e, (((1,), (1,)), ((), ())),
                          preferred_element_type=jnp.float32)
    buf[slot] = acc + b_ref[...]
    _copy(slot, v).start()

    @pl.when(v == _GV - 1)
    def _():
        for s in range(_NBUF):
            _copy(s, v).wait()


_proj = pl.pallas_call(
    _proj_body,
    grid=(_GV,),
    in_specs=[
        pl.BlockSpec((_BV, _D), lambda v: (v, 0)),
        pl.BlockSpec(memory_space=pltpu.VMEM),
        pl.BlockSpec((_BV, 1), lambda v: (v, 0)),
    ],
    out_specs=pl.BlockSpec(memory_space=pltpu.HBM),
    out_shape=jax.ShapeDtypeStruct((_V, _B), jnp.float32),
    scratch_shapes=[
        pltpu.VMEM((_NBUF, _BV, _B), jnp.float32),
        pltpu.SemaphoreType.DMA((_NBUF,)),
    ],
    compiler_params=pltpu.CompilerParams(
        dimension_semantics=("arbitrary",),
    ),
)


def kernel(docs, doc_embeddings, W, b):
    emb = _gather(docs.astype(jnp.int32), doc_embeddings)
    logits_t = _proj(W, emb, b.reshape(_V, 1))
    return logits_t.T


# speedup vs baseline: 1.1033x; 1.1033x over previous
import jax
import jax.numpy as jnp
from jax import lax
from jax.experimental import pallas as pl
from jax.experimental.pallas import tpu as pltpu
from jax.experimental.pallas import tpu_sc as plsc

_B = 4096
_D = 128
_V = 100000

_info = plsc.get_sparse_core_info()
_NC, _NS = _info.num_cores, _info.num_subcores
_NW = _NC * _NS
_BPW = _B // _NW

_BV = 1000
_GV = _V // _BV


def _gather_body(idx_hbm, table_hbm, out_hbm, idx_v, rows_v, sem):
    wid = lax.axis_index("s") * _NC + lax.axis_index("c")
    base = wid * _BPW
    pltpu.sync_copy(idx_hbm.at[pl.ds(base, _BPW)], idx_v)
    pltpu.async_copy(table_hbm.at[idx_v], rows_v, sem).wait()
    pltpu.sync_copy(rows_v, out_hbm.at[pl.ds(base, _BPW)])


_gather = pl.kernel(
    _gather_body,
    out_type=jax.ShapeDtypeStruct((_B, _D), jnp.float32),
    mesh=plsc.VectorSubcoreMesh(core_axis_name="c", subcore_axis_name="s"),
    scratch_types=[
        pltpu.VMEM((_BPW,), jnp.int32),
        pltpu.VMEM((_BPW, _D), jnp.float32),
        pltpu.SemaphoreType.DMA,
    ],
)


_NBUF = 3


def _proj_body(w_ref, emb_ref, b_ref, out_ref, buf, sems):
    v = pl.program_id(0)
    slot = lax.rem(v, _NBUF)

    def _copy(s, vv):
        return pltpu.make_async_copy(
            buf.at[s],
            out_ref.at[pl.ds(vv * _BV, _BV), :],
            sems.at[s],
        )

    @pl.when(v >= _NBUF)
    def _():
        _copy(slot, v).wait()

    w = w_ref[...].astype(jnp.bfloat16)
    e = emb_ref[...].astype(jnp.bfloat16)
    acc = lax.dot_general(w, e, (((1,), (1,)), ((), ())),
                          preferred_element_type=jnp.float32)
    brow = b_ref[0].astype(jnp.bfloat16)
    ones = jnp.ones((1, _B), jnp.bfloat16)
    bias = lax.dot_general(brow, ones, (((0,), (0,)), ((), ())),
                           preferred_element_type=jnp.float32)
    buf[slot] = acc + bias
    _copy(slot, v).start()

    @pl.when(v == _GV - 1)
    def _():
        for s in range(_NBUF):
            _copy(s, v).wait()


_proj = pl.pallas_call(
    _proj_body,
    grid=(_GV,),
    in_specs=[
        pl.BlockSpec((_BV, _D), lambda v: (v, 0)),
        pl.BlockSpec(memory_space=pltpu.VMEM),
        pl.BlockSpec((1, 1, _BV), lambda v: (v, 0, 0)),
    ],
    out_specs=pl.BlockSpec(memory_space=pltpu.HBM),
    out_shape=jax.ShapeDtypeStruct((_V, _B), jnp.float32),
    scratch_shapes=[
        pltpu.VMEM((_NBUF, _BV, _B), jnp.float32),
        pltpu.SemaphoreType.DMA((_NBUF,)),
    ],
    compiler_params=pltpu.CompilerParams(
        dimension_semantics=("arbitrary",),
    ),
)


def kernel(docs, doc_embeddings, W, b):
    emb = _gather(docs.astype(jnp.int32), doc_embeddings)
    logits_t = _proj(W, emb, b.reshape(_GV, 1, _BV))
    return logits_t.T


# speedup vs baseline: 1.1222x; 1.0172x over previous
import jax
import jax.numpy as jnp
from jax import lax
from jax.experimental import pallas as pl
from jax.experimental.pallas import tpu as pltpu
from jax.experimental.pallas import tpu_sc as plsc

_B = 4096
_D = 128
_V = 100000

_info = plsc.get_sparse_core_info()
_NC, _NS = _info.num_cores, _info.num_subcores
_NW = _NC * _NS
_BPW = _B // _NW

_BV = 1000
_GV = _V // _BV


def _gather_body(idx_hbm, table_hbm, out_hbm, idx_v, rows_v, sem):
    wid = lax.axis_index("s") * _NC + lax.axis_index("c")
    base = wid * _BPW
    pltpu.sync_copy(idx_hbm.at[pl.ds(base, _BPW)], idx_v)
    pltpu.async_copy(table_hbm.at[idx_v], rows_v, sem).wait()
    pltpu.sync_copy(rows_v, out_hbm.at[pl.ds(base, _BPW)])


_gather = pl.kernel(
    _gather_body,
    out_type=jax.ShapeDtypeStruct((_B, _D), jnp.float32),
    mesh=plsc.VectorSubcoreMesh(core_axis_name="c", subcore_axis_name="s"),
    scratch_types=[
        pltpu.VMEM((_BPW,), jnp.int32),
        pltpu.VMEM((_BPW, _D), jnp.float32),
        pltpu.SemaphoreType.DMA,
    ],
)


def _proj_body(w_ref, emb_ref, b_ref, out_ref):
    w = w_ref[...].astype(jnp.bfloat16)
    e = emb_ref[...].astype(jnp.bfloat16)
    acc = lax.dot_general(w, e, (((1,), (1,)), ((), ())),
                          preferred_element_type=jnp.float32)
    brow = b_ref[0].astype(jnp.bfloat16)
    ones = jnp.ones((1, _B), jnp.bfloat16)
    bias = lax.dot_general(brow, ones, (((0,), (0,)), ((), ())),
                           preferred_element_type=jnp.float32)
    out_ref[...] = acc + bias


_proj = pl.pallas_call(
    _proj_body,
    grid=(_GV,),
    in_specs=[
        pl.BlockSpec((_BV, _D), lambda v: (v, 0)),
        pl.BlockSpec(memory_space=pltpu.VMEM),
        pl.BlockSpec((1, 1, _BV), lambda v: (v, 0, 0)),
    ],
    out_specs=pl.BlockSpec((_BV, _B), lambda v: (v, 0)),
    out_shape=jax.ShapeDtypeStruct((_V, _B), jnp.float32),
    compiler_params=pltpu.CompilerParams(
        dimension_semantics=("arbitrary",),
    ),
)


def kernel(docs, doc_embeddings, W, b):
    emb = _gather(docs.astype(jnp.int32), doc_embeddings)
    logits_t = _proj(W, emb, b.reshape(_GV, 1, _BV))
    return logits_t.T
